# baseline (speedup 1.0000x reference)
import functools

import jax
import jax.numpy as jnp
from jax import lax
from jax.experimental import pallas as pl
from jax.experimental.pallas import tpu as pltpu
from jax.experimental.pallas import tpu_sc as plsc

TOP_K = 50
MIN_KEEP = 5
D_MODEL = 2048
VOCAB = 100000
BATCH = 8

V_BLK = 2000
N_BLK = VOCAB // V_BLK
V_PAD = 2048

L = 16
BPW = 13
W_WORDS = BPW * V_PAD
CAP = 1024
CBUF = 1152
MCAP = 4 * CBUF
MCHUNKS = MCAP // L
CMAXPAD = MCHUNKS
NEG = float("-inf")
POS = float("inf")
BIG = 2**31 - 1


def _logits_body(hs_ref, g_ref, b_ref, w_ref, out_ref):
    x = hs_ref[...]
    mean = jnp.mean(x, axis=-1, keepdims=True)
    var = jnp.mean((x - mean) ** 2, axis=-1, keepdims=True)
    h = (x - mean) * lax.rsqrt(var + 1e-5) * g_ref[...][None, :] + b_ref[...][None, :]
    w = w_ref[...]
    res = lax.dot_general(
        h, w, (((1,), (1,)), ((), ())), preferred_element_type=jnp.float32
    )
    out_ref[0] = jnp.concatenate(
        [res, jnp.full((BATCH, V_PAD - V_BLK), NEG, jnp.float32)], axis=1)


def _scalar(x):
    return jnp.max(x) if getattr(x, "ndim", 0) else x


def _sc_body(logits_hbm, tp_hbm, tm_hbm, probs_hbm, tok_hbm,
             staged, valsb, idxsb, merged_v, merged_i, cntb, cmax,
             topv, topi, vec16f, vec16i, mx4, pt16, tm16,
             sh_mx, sh_cnt, sh_v, sh_i, sem):
    c = lax.axis_index("c")
    s = lax.axis_index("s")
    rloc = lax.rem(s, 4)
    q = lax.div(s, 4)
    r = 4 * c + rloc
    lane = lax.iota(jnp.int32, 16)

    start = 13 * q - jnp.maximum(0, q - 2)
    end = 13 * (q + 1) - jnp.maximum(0, q - 1)

    descs = []
    for j in range(BPW):
        b = jnp.minimum(start + j, N_BLK - 1)
        descs.append(pltpu.async_copy(
            logits_hbm.at[pl.ds((b * BATCH + r) * V_PAD, V_PAD)],
            staged.at[pl.ds(j * V_PAD, V_PAD)], sem))
    for d in descs:
        d.wait()

    mvec = jnp.full((L,), POS, jnp.float32)
    for j in range(BPW):
        def mbody(cc, acc):
            return jnp.maximum(acc, staged[pl.ds(j * V_PAD + cc * L, L)])
        macc = lax.fori_loop(0, V_PAD // L, mbody, jnp.full((L,), NEG, jnp.float32))
        mvec = jnp.where(lane == j, jnp.full((L,), jnp.max(macc)), mvec)
    vec16f[pl.ds(0, L)] = mvec
    pltpu.sync_copy(vec16f, sh_mx.at[s])
    plsc.subcore_barrier()

    for qq in range(4):
        pltpu.sync_copy(sh_mx.at[4 * qq + rloc], mx4.at[pl.ds(qq * 128, 128)])
    tmin = jnp.full((L,), POS, jnp.float32)
    for qq in range(4):
        tmin = jnp.minimum(tmin, mx4[pl.ds(qq * 128, L)])
    tsp = jnp.full((L,), jnp.min(tmin))

    def cbody(k, cnt):
        v = staged[pl.ds(k * L, L)]
        j = lax.div(k, V_PAD // L)
        gidx = jnp.full((L,), start * V_BLK + k * L - j * (V_PAD - V_BLK),
                        jnp.int32) + lane
        mask = (v >= tsp) & (gidx < jnp.full((L,), end * V_BLK, jnp.int32))
        mask = mask & (jnp.full((L,), cnt, jnp.int32) < CAP)
        plsc.store_compressed(valsb.at[pl.ds(cnt, L)], v, mask=mask)
        plsc.store_compressed(idxsb.at[pl.ds(cnt, L)], gidx, mask=mask)
        return cnt + _scalar(plsc.all_reduce_population_count(mask))

    cnt = lax.fori_loop(0, W_WORDS // L, cbody, jnp.int32(0))

    vec16i[pl.ds(0, L)] = jnp.full((L,), cnt, jnp.int32)
    pltpu.sync_copy(vec16i, sh_cnt.at[s])
    pltpu.sync_copy(valsb, sh_v.at[s])
    pltpu.sync_copy(idxsb, sh_i.at[s])
    plsc.subcore_barrier()

    @pl.when(s < 4)
    def _leader():
        for qq in range(4):
            pltpu.sync_copy(sh_v.at[4 * qq + s], merged_v.at[pl.ds(qq * CBUF, CBUF)])
            pltpu.sync_copy(sh_i.at[4 * qq + s], merged_i.at[pl.ds(qq * CBUF, CBUF)])
            pltpu.sync_copy(sh_cnt.at[4 * qq + s], cntb.at[pl.ds(qq * 128, 128)])
        pltpu.sync_copy(tp_hbm, pt16)
        pltpu.sync_copy(tm_hbm, tm16)

        def clean(t, _):
            qq = lax.div(t, CBUF // L)
            cq = cntb[pl.ds(qq * 128, L)]
            pos = jnp.full((L,), (t - qq * (CBUF // L)) * L, jnp.int32) + lane
            v = jnp.where(pos < cq, merged_v[pl.ds(t * L, L)], NEG)
            merged_v[pl.ds(t * L, L)] = v
            plsc.store_scatter(cmax, [jnp.full((L,), t, jnp.int32)],
                               jnp.full((L,), jnp.max(v)), mask=lane == 0)
            return 0

        lax.fori_loop(0, MCHUNKS, clean, 0)

        for cc in range(8):
            topv[pl.ds(cc * L, L)] = jnp.full((L,), NEG, jnp.float32)
            topi[pl.ds(cc * L, L)] = jnp.full((L,), jnp.int32(0))

        def extract(i, _):
            macc = jnp.full((L,), NEG, jnp.float32)
            for tt in range(CMAXPAD // L):
                macc = jnp.maximum(macc, cmax[pl.ds(tt * L, L)])
            m = jnp.max(macc)
            msp = jnp.full((L,), m)
            cidx = jnp.full((L,), BIG, jnp.int32)
            for tt in range(CMAXPAD // L):
                cm = cmax[pl.ds(tt * L, L)]
                pos = jnp.full((L,), tt * L, jnp.int32) + lane
                cidx = jnp.minimum(cidx, jnp.where(cm == msp, pos, BIG))
            cstar = jnp.min(cidx)
            v = merged_v[pl.ds(cstar * L, L)]
            lanei = _scalar(plsc.all_reduce_ffs(v == msp))
            lsp = jnp.full((L,), lanei, jnp.int32)
            gv = merged_i[pl.ds(cstar * L, L)]
            tok = jnp.min(jnp.where(lane == lsp, gv, BIG))
            v2 = jnp.where(lane == lsp, NEG, v)
            merged_v[pl.ds(cstar * L, L)] = v2
            plsc.store_scatter(cmax, [jnp.full((L,), cstar, jnp.int32)],
                               jnp.full((L,), jnp.max(v2)), mask=lane == 0)
            plsc.store_scatter(topv, [jnp.full((L,), i, jnp.int32)],
                               msp, mask=lane == 0)
            plsc.store_scatter(topi, [jnp.full((L,), i, jnp.int32)],
                               jnp.full((L,), tok), mask=lane == 0)
            return 0

        lax.fori_loop(0, TOP_K, extract, 0)

        tpv = pt16[pl.ds(0, L)]
        tmv = tm16[pl.ds(0, L)]
        tl = [topv[pl.ds(cc * L, L)] / tmv for cc in range(4)]
        m1 = jnp.max(jnp.maximum(jnp.maximum(tl[0], tl[1]),
                                 jnp.maximum(tl[2], tl[3])))
        poss = [jnp.full((L,), cc * L, jnp.int32) + lane for cc in range(4)]
        e = [jnp.where(poss[cc] < TOP_K,
                       jnp.exp(tl[cc] - jnp.full((L,), m1)),
                       jnp.float32(0.0)) for cc in range(4)]
        ssum = jnp.max(jnp.full((L,), jnp.sum(e[0]) + jnp.sum(e[1])
                                      + jnp.sum(e[2]) + jnp.sum(e[3])))
        fl = []
        car = jnp.float32(0.0)
        for cc in range(4):
            p = e[cc] / jnp.full((L,), ssum)
            cu = plsc.cumsum(p) + jnp.full((L,), car)
            car = car + jnp.sum(p)
            keep = (cu < tpv) | (poss[cc] < MIN_KEEP)
            fl.append(jnp.where(keep, tl[cc], jnp.float32(-1000.0)))
        m2 = jnp.max(jnp.maximum(jnp.maximum(fl[0], fl[1]),
                                 jnp.maximum(fl[2], fl[3])))
        e2 = [jnp.where(poss[cc] < TOP_K,
                        jnp.exp(fl[cc] - jnp.full((L,), m2)),
                        jnp.float32(0.0)) for cc in range(4)]
        s2 = jnp.max(jnp.full((L,), jnp.sum(e2[0]) + jnp.sum(e2[1])
                                    + jnp.sum(e2[2]) + jnp.sum(e2[3])))
        for cc in range(4):
            topv[pl.ds(cc * L, L)] = e2[cc] / jnp.full((L,), s2)
        pltpu.sync_copy(topv, probs_hbm.at[r])
        pltpu.sync_copy(topi, tok_hbm.at[r])


def _make_sc_sampler():
    mesh = plsc.VectorSubcoreMesh(core_axis_name="c", subcore_axis_name="s")

    return pl.kernel(
        _sc_body,
        out_type=[
            jax.ShapeDtypeStruct((BATCH, 128), jnp.float32),
            jax.ShapeDtypeStruct((BATCH, 128), jnp.int32),
        ],
        mesh=mesh,
        compiler_params=pltpu.CompilerParams(needs_layout_passes=False),
        scratch_types=[
            pltpu.VMEM((W_WORDS,), jnp.float32),
            pltpu.VMEM((CBUF,), jnp.float32),
            pltpu.VMEM((CBUF,), jnp.int32),
            pltpu.VMEM((MCAP,), jnp.float32),
            pltpu.VMEM((MCAP,), jnp.int32),
            pltpu.VMEM((512,), jnp.int32),
            pltpu.VMEM((CMAXPAD,), jnp.float32),
            pltpu.VMEM((128,), jnp.float32),
            pltpu.VMEM((128,), jnp.int32),
            pltpu.VMEM((128,), jnp.float32),
            pltpu.VMEM((128,), jnp.int32),
            pltpu.VMEM((512,), jnp.float32),
            pltpu.VMEM((128,), jnp.float32),
            pltpu.VMEM((128,), jnp.float32),
            pltpu.VMEM_SHARED((16, 128), jnp.float32),
            pltpu.VMEM_SHARED((16, 128), jnp.int32),
            pltpu.VMEM_SHARED((16, CBUF), jnp.float32),
            pltpu.VMEM_SHARED((16, CBUF), jnp.int32),
            pltpu.SemaphoreType.DMA,
        ],
    )


@functools.partial(jax.jit, static_argnames=("interpret",))
def kernel(hidden_states, top_p, temperature, ln_gamma, ln_beta, lm_head_w,
           interpret=False):
    logits = pl.pallas_call(
        _logits_body,
        grid=(N_BLK,),
        in_specs=[
            pl.BlockSpec((BATCH, D_MODEL), lambda i: (0, 0)),
            pl.BlockSpec((D_MODEL,), lambda i: (0,)),
            pl.BlockSpec((D_MODEL,), lambda i: (0,)),
            pl.BlockSpec((V_BLK, D_MODEL), lambda i: (i, 0)),
        ],
        out_specs=pl.BlockSpec((1, BATCH, V_PAD), lambda i: (i, 0, 0)),
        out_shape=jax.ShapeDtypeStruct((N_BLK, BATCH, V_PAD), jnp.float32),
        interpret=interpret,
    )(hidden_states, ln_gamma, ln_beta, lm_head_w)

    tp16 = jnp.broadcast_to(top_p.astype(jnp.float32), (128,))
    tm16 = jnp.broadcast_to(temperature.astype(jnp.float32), (128,))
    probs64, tok64 = _make_sc_sampler()(logits.reshape(-1), tp16, tm16)
    return probs64[:, :TOP_K], tok64[:, :TOP_K]


# speedup vs baseline: 1.0333x; 1.0333x over previous
import functools

import jax
import jax.numpy as jnp
from jax import lax
from jax.experimental import pallas as pl
from jax.experimental.pallas import tpu as pltpu
from jax.experimental.pallas import tpu_sc as plsc

TOP_K = 50
MIN_KEEP = 5
D_MODEL = 2048
VOCAB = 100000
BATCH = 8

V_BLK = 2000
N_BLK = VOCAB // V_BLK
V_PAD = 2048

L = 16
BPW = 13
W_WORDS = BPW * V_PAD
CAP = 1024
CBUF = 1152
MCAP = 4 * CBUF
MCHUNKS = MCAP // L
CMAXPAD = MCHUNKS
NEG = float("-inf")
POS = float("inf")
BIG = 2**31 - 1


def _logits_body(hs_ref, g_ref, b_ref, w_ref, out_ref):
    x = hs_ref[...]
    mean = jnp.mean(x, axis=-1, keepdims=True)
    var = jnp.mean((x - mean) ** 2, axis=-1, keepdims=True)
    h = (x - mean) * lax.rsqrt(var + 1e-5) * g_ref[...][None, :] + b_ref[...][None, :]
    w = w_ref[...]
    res = lax.dot_general(
        h, w, (((1,), (1,)), ((), ())), preferred_element_type=jnp.float32
    )
    padded = jnp.concatenate(
        [res, jnp.full((BATCH, V_PAD - V_BLK), NEG, jnp.float32)], axis=1)
    out_ref[...] = padded.reshape(BATCH * V_PAD)


def _scalar(x):
    return jnp.max(x) if getattr(x, "ndim", 0) else x


def _sc_body(logits_hbm, tp_hbm, tm_hbm, probs_hbm, tok_hbm,
             staged, valsb, idxsb, merged_v, merged_i, cntb, cmax,
             topv, topi, vec16f, vec16i, mx4, pt16, tm16,
             sh_mx, sh_cnt, sh_v, sh_i, sem):
    c = lax.axis_index("c")
    s = lax.axis_index("s")
    rloc = lax.rem(s, 4)
    q = lax.div(s, 4)
    r = 4 * c + rloc
    lane = lax.iota(jnp.int32, 16)

    start = 13 * q - jnp.maximum(0, q - 2)
    end = 13 * (q + 1) - jnp.maximum(0, q - 1)

    descs = []
    for j in range(BPW):
        b = jnp.minimum(start + j, N_BLK - 1)
        descs.append(pltpu.async_copy(
            logits_hbm.at[pl.ds((b * BATCH + r) * V_PAD, V_PAD)],
            staged.at[pl.ds(j * V_PAD, V_PAD)], sem))
    for d in descs:
        d.wait()

    mvec = jnp.full((L,), POS, jnp.float32)
    for j in range(BPW):
        def mbody(cc, acc):
            for u in range(8):
                acc = jnp.maximum(
                    acc, staged[pl.ds(j * V_PAD + (cc * 8 + u) * L, L)])
            return acc
        macc = lax.fori_loop(0, V_PAD // L // 8, mbody,
                             jnp.full((L,), NEG, jnp.float32))
        mvec = jnp.where(lane == j, jnp.full((L,), jnp.max(macc)), mvec)
    vec16f[pl.ds(0, L)] = mvec
    pltpu.sync_copy(vec16f, sh_mx.at[s])
    plsc.subcore_barrier()

    for qq in range(4):
        pltpu.sync_copy(sh_mx.at[4 * qq + rloc], mx4.at[pl.ds(qq * 128, 128)])
    tmin = jnp.full((L,), POS, jnp.float32)
    for qq in range(4):
        tmin = jnp.minimum(tmin, mx4[pl.ds(qq * 128, L)])
    tsp = jnp.full((L,), jnp.min(tmin))

    def cbody(k8, cnt):
        for u in range(8):
            k = k8 * 8 + u
            v = staged[pl.ds(k * L, L)]
            j = lax.div(k, V_PAD // L)
            gidx = jnp.full((L,), start * V_BLK + k * L - j * (V_PAD - V_BLK),
                            jnp.int32) + lane
            mask = (v >= tsp) & (gidx < jnp.full((L,), end * V_BLK, jnp.int32))
            mask = mask & (jnp.full((L,), cnt, jnp.int32) < CAP)
            plsc.store_compressed(valsb.at[pl.ds(cnt, L)], v, mask=mask)
            plsc.store_compressed(idxsb.at[pl.ds(cnt, L)], gidx, mask=mask)
            cnt = cnt + _scalar(plsc.all_reduce_population_count(mask))
        return cnt

    cnt = lax.fori_loop(0, W_WORDS // L // 8, cbody, jnp.int32(0))

    vec16i[pl.ds(0, L)] = jnp.full((L,), cnt, jnp.int32)
    pltpu.sync_copy(vec16i, sh_cnt.at[s])
    pltpu.sync_copy(valsb, sh_v.at[s])
    pltpu.sync_copy(idxsb, sh_i.at[s])
    plsc.subcore_barrier()

    @pl.when(s < 4)
    def _leader():
        for qq in range(4):
            pltpu.sync_copy(sh_v.at[4 * qq + s], merged_v.at[pl.ds(qq * CBUF, CBUF)])
            pltpu.sync_copy(sh_i.at[4 * qq + s], merged_i.at[pl.ds(qq * CBUF, CBUF)])
            pltpu.sync_copy(sh_cnt.at[4 * qq + s], cntb.at[pl.ds(qq * 128, 128)])
        pltpu.sync_copy(tp_hbm, pt16)
        pltpu.sync_copy(tm_hbm, tm16)

        def clean(t, _):
            qq = lax.div(t, CBUF // L)
            cq = cntb[pl.ds(qq * 128, L)]
            pos = jnp.full((L,), (t - qq * (CBUF // L)) * L, jnp.int32) + lane
            v = jnp.where(pos < cq, merged_v[pl.ds(t * L, L)], NEG)
            merged_v[pl.ds(t * L, L)] = v
            plsc.store_scatter(cmax, [jnp.full((L,), t, jnp.int32)],
                               jnp.full((L,), jnp.max(v)), mask=lane == 0)
            return 0

        lax.fori_loop(0, MCHUNKS, clean, 0)

        for cc in range(8):
            topv[pl.ds(cc * L, L)] = jnp.full((L,), NEG, jnp.float32)
            topi[pl.ds(cc * L, L)] = jnp.full((L,), jnp.int32(0))

        def extract(i, _):
            macc = jnp.full((L,), NEG, jnp.float32)
            for tt in range(CMAXPAD // L):
                macc = jnp.maximum(macc, cmax[pl.ds(tt * L, L)])
            m = jnp.max(macc)
            msp = jnp.full((L,), m)
            cidx = jnp.full((L,), BIG, jnp.int32)
            for tt in range(CMAXPAD // L):
                cm = cmax[pl.ds(tt * L, L)]
                pos = jnp.full((L,), tt * L, jnp.int32) + lane
                cidx = jnp.minimum(cidx, jnp.where(cm == msp, pos, BIG))
            cstar = jnp.min(cidx)
            v = merged_v[pl.ds(cstar * L, L)]
            lanei = _scalar(plsc.all_reduce_ffs(v == msp))
            lsp = jnp.full((L,), lanei, jnp.int32)
            gv = merged_i[pl.ds(cstar * L, L)]
            tok = jnp.min(jnp.where(lane == lsp, gv, BIG))
            v2 = jnp.where(lane == lsp, NEG, v)
            merged_v[pl.ds(cstar * L, L)] = v2
            plsc.store_scatter(cmax, [jnp.full((L,), cstar, jnp.int32)],
                               jnp.full((L,), jnp.max(v2)), mask=lane == 0)
            plsc.store_scatter(topv, [jnp.full((L,), i, jnp.int32)],
                               msp, mask=lane == 0)
            plsc.store_scatter(topi, [jnp.full((L,), i, jnp.int32)],
                               jnp.full((L,), tok), mask=lane == 0)
            return 0

        lax.fori_loop(0, TOP_K, extract, 0)

        tpv = pt16[pl.ds(0, L)]
        tmv = tm16[pl.ds(0, L)]
        tl = [topv[pl.ds(cc * L, L)] / tmv for cc in range(4)]
        m1 = jnp.max(jnp.maximum(jnp.maximum(tl[0], tl[1]),
                                 jnp.maximum(tl[2], tl[3])))
        poss = [jnp.full((L,), cc * L, jnp.int32) + lane for cc in range(4)]
        e = [jnp.where(poss[cc] < TOP_K,
                       jnp.exp(tl[cc] - jnp.full((L,), m1)),
                       jnp.float32(0.0)) for cc in range(4)]
        ssum = jnp.max(jnp.full((L,), jnp.sum(e[0]) + jnp.sum(e[1])
                                      + jnp.sum(e[2]) + jnp.sum(e[3])))
        fl = []
        car = jnp.float32(0.0)
        for cc in range(4):
            p = e[cc] / jnp.full((L,), ssum)
            cu = plsc.cumsum(p) + jnp.full((L,), car)
            car = car + jnp.sum(p)
            keep = (cu < tpv) | (poss[cc] < MIN_KEEP)
            fl.append(jnp.where(keep, tl[cc], jnp.float32(-1000.0)))
        m2 = jnp.max(jnp.maximum(jnp.maximum(fl[0], fl[1]),
                                 jnp.maximum(fl[2], fl[3])))
        e2 = [jnp.where(poss[cc] < TOP_K,
                        jnp.exp(fl[cc] - jnp.full((L,), m2)),
                        jnp.float32(0.0)) for cc in range(4)]
        s2 = jnp.max(jnp.full((L,), jnp.sum(e2[0]) + jnp.sum(e2[1])
                                    + jnp.sum(e2[2]) + jnp.sum(e2[3])))
        for cc in range(4):
            topv[pl.ds(cc * L, L)] = e2[cc] / jnp.full((L,), s2)
        pltpu.sync_copy(topv, probs_hbm.at[r])
        pltpu.sync_copy(topi, tok_hbm.at[r])


def _make_sc_sampler():
    mesh = plsc.VectorSubcoreMesh(core_axis_name="c", subcore_axis_name="s")

    return pl.kernel(
        _sc_body,
        out_type=[
            jax.ShapeDtypeStruct((BATCH, 128), jnp.float32),
            jax.ShapeDtypeStruct((BATCH, 128), jnp.int32),
        ],
        mesh=mesh,
        compiler_params=pltpu.CompilerParams(needs_layout_passes=False),
        scratch_types=[
            pltpu.VMEM((W_WORDS,), jnp.float32),
            pltpu.VMEM((CBUF,), jnp.float32),
            pltpu.VMEM((CBUF,), jnp.int32),
            pltpu.VMEM((MCAP,), jnp.float32),
            pltpu.VMEM((MCAP,), jnp.int32),
            pltpu.VMEM((512,), jnp.int32),
            pltpu.VMEM((CMAXPAD,), jnp.float32),
            pltpu.VMEM((128,), jnp.float32),
            pltpu.VMEM((128,), jnp.int32),
            pltpu.VMEM((128,), jnp.float32),
            pltpu.VMEM((128,), jnp.int32),
            pltpu.VMEM((512,), jnp.float32),
            pltpu.VMEM((128,), jnp.float32),
            pltpu.VMEM((128,), jnp.float32),
            pltpu.VMEM_SHARED((16, 128), jnp.float32),
            pltpu.VMEM_SHARED((16, 128), jnp.int32),
            pltpu.VMEM_SHARED((16, CBUF), jnp.float32),
            pltpu.VMEM_SHARED((16, CBUF), jnp.int32),
            pltpu.SemaphoreType.DMA,
        ],
    )


@functools.partial(jax.jit, static_argnames=("interpret",))
def kernel(hidden_states, top_p, temperature, ln_gamma, ln_beta, lm_head_w,
           interpret=False):
    logits = pl.pallas_call(
        _logits_body,
        grid=(N_BLK,),
        in_specs=[
            pl.BlockSpec((BATCH, D_MODEL), lambda i: (0, 0)),
            pl.BlockSpec((D_MODEL,), lambda i: (0,)),
            pl.BlockSpec((D_MODEL,), lambda i: (0,)),
            pl.BlockSpec((V_BLK, D_MODEL), lambda i: (i, 0)),
        ],
        out_specs=pl.BlockSpec((BATCH * V_PAD,), lambda i: (i,)),
        out_shape=jax.ShapeDtypeStruct((N_BLK * BATCH * V_PAD,), jnp.float32),
        interpret=interpret,
    )(hidden_states, ln_gamma, ln_beta, lm_head_w)

    tp16 = jnp.broadcast_to(top_p.astype(jnp.float32), (128,))
    tm16 = jnp.broadcast_to(temperature.astype(jnp.float32), (128,))
    probs64, tok64 = _make_sc_sampler()(logits, tp16, tm16)
    return probs64[:, :TOP_K], tok64[:, :TOP_K]


# speedup vs baseline: 1.1204x; 1.0843x over previous
import functools

import jax
import jax.numpy as jnp
from jax import lax
from jax.experimental import pallas as pl
from jax.experimental.pallas import tpu as pltpu
from jax.experimental.pallas import tpu_sc as plsc

TOP_K = 50
MIN_KEEP = 5
D_MODEL = 2048
VOCAB = 100000
BATCH = 8

V_BLK = 2000
N_BLK = VOCAB // V_BLK
V_PAD = 2048

L = 16
BPW = 13
W_WORDS = BPW * V_PAD
CAP = 512
CBUF = 640
MCAP = 4 * CBUF
MCHUNKS = MCAP // L
CMAXPAD = MCHUNKS
NEG = float("-inf")
POS = float("inf")
BIG = 2**31 - 1


def _logits_body(hs_ref, g_ref, b_ref, w_ref, out_ref):
    x = hs_ref[...]
    mean = jnp.mean(x, axis=-1, keepdims=True)
    var = jnp.mean((x - mean) ** 2, axis=-1, keepdims=True)
    h = (x - mean) * lax.rsqrt(var + 1e-5) * g_ref[...][None, :] + b_ref[...][None, :]
    w = w_ref[...]
    res = lax.dot_general(
        h, w, (((1,), (1,)), ((), ())), preferred_element_type=jnp.float32
    )
    padded = jnp.concatenate(
        [res, jnp.full((BATCH, V_PAD - V_BLK), NEG, jnp.float32)], axis=1)
    out_ref[...] = padded.reshape(BATCH * V_PAD)


def _scalar(x):
    return jnp.max(x) if getattr(x, "ndim", 0) else x


def _sc_body(logits_hbm, tp_hbm, tm_hbm, probs_hbm, tok_hbm,
             staged, valsb, idxsb, merged_v, merged_i, cntb, cmax,
             topv, topi, vec16f, vec16i, mx4, pt16, tm16,
             sh_mx, sh_cnt, sh_v, sh_i, sem):
    c = lax.axis_index("c")
    s = lax.axis_index("s")
    rloc = lax.rem(s, 4)
    q = lax.div(s, 4)
    r = 4 * c + rloc
    lane = lax.iota(jnp.int32, 16)

    start = 13 * q - jnp.maximum(0, q - 2)
    end = 13 * (q + 1) - jnp.maximum(0, q - 1)

    descs = []
    for j in range(BPW):
        b = jnp.minimum(start + j, N_BLK - 1)
        descs.append(pltpu.async_copy(
            logits_hbm.at[pl.ds((b * BATCH + r) * V_PAD, V_PAD)],
            staged.at[pl.ds(j * V_PAD, V_PAD)], sem))
    for d in descs:
        d.wait()

    mvec = jnp.full((L,), POS, jnp.float32)
    for j in range(BPW):
        def mbody(cc, acc):
            for u in range(8):
                acc = jnp.maximum(
                    acc, staged[pl.ds(j * V_PAD + (cc * 8 + u) * L, L)])
            return acc
        macc = lax.fori_loop(0, V_PAD // L // 8, mbody,
                             jnp.full((L,), NEG, jnp.float32))
        mvec = jnp.where(lane == j, jnp.full((L,), jnp.max(macc)), mvec)
    vec16f[pl.ds(0, L)] = mvec
    pltpu.sync_copy(vec16f, sh_mx.at[s])
    plsc.subcore_barrier()

    for qq in range(4):
        pltpu.sync_copy(sh_mx.at[4 * qq + rloc], mx4.at[pl.ds(qq * 128, 128)])
    tmin = jnp.full((L,), POS, jnp.float32)
    for qq in range(4):
        tmin = jnp.minimum(tmin, mx4[pl.ds(qq * 128, L)])
    tsp = jnp.full((L,), jnp.min(tmin))

    def cbody(k8, cnt):
        vs, gs, ms, pcs = [], [], [], []
        for u in range(8):
            k = k8 * 8 + u
            v = staged[pl.ds(k * L, L)]
            j = lax.div(k, V_PAD // L)
            gidx = jnp.full((L,), start * V_BLK + k * L - j * (V_PAD - V_BLK),
                            jnp.int32) + lane
            mask = (v >= tsp) & (gidx < jnp.full((L,), end * V_BLK, jnp.int32))
            vs.append(v)
            gs.append(gidx)
            ms.append(mask)
            pcs.append(_scalar(plsc.all_reduce_population_count(mask)))
        for u in range(8):
            mask = ms[u] & (jnp.full((L,), cnt, jnp.int32) < CAP)
            plsc.store_compressed(valsb.at[pl.ds(cnt, L)], vs[u], mask=mask)
            plsc.store_compressed(idxsb.at[pl.ds(cnt, L)], gs[u], mask=mask)
            cnt = cnt + pcs[u]
        return cnt

    cnt = lax.fori_loop(0, W_WORDS // L // 8, cbody, jnp.int32(0))
    cnt = jnp.minimum(cnt, CAP)

    vec16i[pl.ds(0, L)] = jnp.full((L,), cnt, jnp.int32)
    pltpu.sync_copy(vec16i, sh_cnt.at[s])
    pltpu.sync_copy(valsb, sh_v.at[s])
    pltpu.sync_copy(idxsb, sh_i.at[s])
    plsc.subcore_barrier()

    @pl.when(s < 4)
    def _leader():
        for qq in range(4):
            pltpu.sync_copy(sh_v.at[4 * qq + s], merged_v.at[pl.ds(qq * CBUF, CBUF)])
            pltpu.sync_copy(sh_i.at[4 * qq + s], merged_i.at[pl.ds(qq * CBUF, CBUF)])
            pltpu.sync_copy(sh_cnt.at[4 * qq + s], cntb.at[pl.ds(qq * 128, 128)])
        pltpu.sync_copy(tp_hbm, pt16)
        pltpu.sync_copy(tm_hbm, tm16)

        def clean(t, _):
            qq = lax.div(t, CBUF // L)
            cq = cntb[pl.ds(qq * 128, L)]
            pos = jnp.full((L,), (t - qq * (CBUF // L)) * L, jnp.int32) + lane
            v = jnp.where(pos < cq, merged_v[pl.ds(t * L, L)], NEG)
            merged_v[pl.ds(t * L, L)] = v
            plsc.store_scatter(cmax, [jnp.full((L,), t, jnp.int32)],
                               jnp.full((L,), jnp.max(v)), mask=lane == 0)
            return 0

        lax.fori_loop(0, MCHUNKS, clean, 0)

        for cc in range(8):
            topv[pl.ds(cc * L, L)] = jnp.full((L,), NEG, jnp.float32)
            topi[pl.ds(cc * L, L)] = jnp.full((L,), jnp.int32(0))

        def extract(i, _):
            macc = jnp.full((L,), NEG, jnp.float32)
            for tt in range(CMAXPAD // L):
                macc = jnp.maximum(macc, cmax[pl.ds(tt * L, L)])
            m = jnp.max(macc)
            msp = jnp.full((L,), m)
            cidx = jnp.full((L,), BIG, jnp.int32)
            for tt in range(CMAXPAD // L):
                cm = cmax[pl.ds(tt * L, L)]
                pos = jnp.full((L,), tt * L, jnp.int32) + lane
                cidx = jnp.minimum(cidx, jnp.where(cm == msp, pos, BIG))
            cstar = jnp.min(cidx)
            v = merged_v[pl.ds(cstar * L, L)]
            lanei = _scalar(plsc.all_reduce_ffs(v == msp))
            lsp = jnp.full((L,), lanei, jnp.int32)
            gv = merged_i[pl.ds(cstar * L, L)]
            tok = jnp.min(jnp.where(lane == lsp, gv, BIG))
            v2 = jnp.where(lane == lsp, NEG, v)
            merged_v[pl.ds(cstar * L, L)] = v2
            plsc.store_scatter(cmax, [jnp.full((L,), cstar, jnp.int32)],
                               jnp.full((L,), jnp.max(v2)), mask=lane == 0)
            plsc.store_scatter(topv, [jnp.full((L,), i, jnp.int32)],
                               msp, mask=lane == 0)
            plsc.store_scatter(topi, [jnp.full((L,), i, jnp.int32)],
                               jnp.full((L,), tok), mask=lane == 0)
            return 0

        lax.fori_loop(0, TOP_K, extract, 0)

        tpv = pt16[pl.ds(0, L)]
        tmv = tm16[pl.ds(0, L)]
        tl = [topv[pl.ds(cc * L, L)] / tmv for cc in range(4)]
        m1 = jnp.max(jnp.maximum(jnp.maximum(tl[0], tl[1]),
                                 jnp.maximum(tl[2], tl[3])))
        poss = [jnp.full((L,), cc * L, jnp.int32) + lane for cc in range(4)]
        e = [jnp.where(poss[cc] < TOP_K,
                       jnp.exp(tl[cc] - jnp.full((L,), m1)),
                       jnp.float32(0.0)) for cc in range(4)]
        ssum = jnp.max(jnp.full((L,), jnp.sum(e[0]) + jnp.sum(e[1])
                                      + jnp.sum(e[2]) + jnp.sum(e[3])))
        fl = []
        car = jnp.float32(0.0)
        for cc in range(4):
            p = e[cc] / jnp.full((L,), ssum)
            cu = plsc.cumsum(p) + jnp.full((L,), car)
            car = car + jnp.sum(p)
            keep = (cu < tpv) | (poss[cc] < MIN_KEEP)
            fl.append(jnp.where(keep, tl[cc], jnp.float32(-1000.0)))
        m2 = jnp.max(jnp.maximum(jnp.maximum(fl[0], fl[1]),
                                 jnp.maximum(fl[2], fl[3])))
        e2 = [jnp.where(poss[cc] < TOP_K,
                        jnp.exp(fl[cc] - jnp.full((L,), m2)),
                        jnp.float32(0.0)) for cc in range(4)]
        s2 = jnp.max(jnp.full((L,), jnp.sum(e2[0]) + jnp.sum(e2[1])
                                    + jnp.sum(e2[2]) + jnp.sum(e2[3])))
        for cc in range(4):
            topv[pl.ds(cc * L, L)] = e2[cc] / jnp.full((L,), s2)
        pltpu.sync_copy(topv, probs_hbm.at[r])
        pltpu.sync_copy(topi, tok_hbm.at[r])


def _make_sc_sampler():
    mesh = plsc.VectorSubcoreMesh(core_axis_name="c", subcore_axis_name="s")

    return pl.kernel(
        _sc_body,
        out_type=[
            jax.ShapeDtypeStruct((BATCH, 128), jnp.float32),
            jax.ShapeDtypeStruct((BATCH, 128), jnp.int32),
        ],
        mesh=mesh,
        compiler_params=pltpu.CompilerParams(needs_layout_passes=False),
        scratch_types=[
            pltpu.VMEM((W_WORDS,), jnp.float32),
            pltpu.VMEM((CBUF,), jnp.float32),
            pltpu.VMEM((CBUF,), jnp.int32),
            pltpu.VMEM((MCAP,), jnp.float32),
            pltpu.VMEM((MCAP,), jnp.int32),
            pltpu.VMEM((512,), jnp.int32),
            pltpu.VMEM((CMAXPAD,), jnp.float32),
            pltpu.VMEM((128,), jnp.float32),
            pltpu.VMEM((128,), jnp.int32),
            pltpu.VMEM((128,), jnp.float32),
            pltpu.VMEM((128,), jnp.int32),
            pltpu.VMEM((512,), jnp.float32),
            pltpu.VMEM((128,), jnp.float32),
            pltpu.VMEM((128,), jnp.float32),
            pltpu.VMEM_SHARED((16, 128), jnp.float32),
            pltpu.VMEM_SHARED((16, 128), jnp.int32),
            pltpu.VMEM_SHARED((16, CBUF), jnp.float32),
            pltpu.VMEM_SHARED((16, CBUF), jnp.int32),
            pltpu.SemaphoreType.DMA,
        ],
    )


@functools.partial(jax.jit, static_argnames=("interpret",))
def kernel(hidden_states, top_p, temperature, ln_gamma, ln_beta, lm_head_w,
           interpret=False):
    logits = pl.pallas_call(
        _logits_body,
        grid=(N_BLK,),
        in_specs=[
            pl.BlockSpec((BATCH, D_MODEL), lambda i: (0, 0)),
            pl.BlockSpec((D_MODEL,), lambda i: (0,)),
            pl.BlockSpec((D_MODEL,), lambda i: (0,)),
            pl.BlockSpec((V_BLK, D_MODEL), lambda i: (i, 0)),
        ],
        out_specs=pl.BlockSpec((BATCH * V_PAD,), lambda i: (i,)),
        out_shape=jax.ShapeDtypeStruct((N_BLK * BATCH * V_PAD,), jnp.float32),
        interpret=interpret,
    )(hidden_states, ln_gamma, ln_beta, lm_head_w)

    tp16 = jnp.broadcast_to(top_p.astype(jnp.float32), (128,))
    tm16 = jnp.broadcast_to(temperature.astype(jnp.float32), (128,))
    probs64, tok64 = _make_sc_sampler()(logits, tp16, tm16)
    return probs64[:, :TOP_K], tok64[:, :TOP_K]
